# Initial kernel scaffold; baseline (speedup 1.0000x reference)
import functools

import jax
import jax.numpy as jnp
from jax import lax
from jax.experimental import pallas as pl
from jax.experimental.pallas import tpu as pltpu
from jax.experimental.pallas import tpu_sc as plsc

_TM = 512
_TK = 2048
_NC = 2
_NS = 16
_NW = _NC * _NS


def _argmin_body(a_ref, bt_ref, a2_ref, b2_ref, out_ref, minv, mini):
    j = pl.program_id(1)
    nj = pl.num_programs(1)

    @pl.when(j == 0)
    def _init():
        minv[...] = jnp.full(minv.shape, jnp.inf, jnp.float32)
        mini[...] = jnp.zeros(mini.shape, jnp.int32)

    ab = jnp.dot(a_ref[...], bt_ref[...], preferred_element_type=jnp.float32)
    d = a2_ref[...] + b2_ref[...] - 2.0 * ab
    m = jnp.min(d, axis=1, keepdims=True)
    iota = lax.broadcasted_iota(jnp.int32, d.shape, 1) + j * _TK
    il = jnp.min(jnp.where(d == m, iota, jnp.int32(2**30)), axis=1,
                 keepdims=True)
    better = m < minv[...]
    mini[...] = jnp.where(better, il, mini[...])
    minv[...] = jnp.where(better, m, minv[...])

    @pl.when(j == nj - 1)
    def _flush():
        out_ref[...] = mini[...]


def _distance_argmin(a_bf, bt_bf, a2, b2):
    n, d = a_bf.shape
    k = bt_bf.shape[1]
    grid = (n // _TM, k // _TK)
    return pl.pallas_call(
        _argmin_body,
        grid=grid,
        in_specs=[
            pl.BlockSpec((_TM, d), lambda i, j: (i, 0)),
            pl.BlockSpec((d, _TK), lambda i, j: (0, j)),
            pl.BlockSpec((_TM, 1), lambda i, j: (i, 0)),
            pl.BlockSpec((1, _TK), lambda i, j: (0, j)),
        ],
        out_specs=pl.BlockSpec((_TM, 1), lambda i, j: (i, 0)),
        out_shape=jax.ShapeDtypeStruct((n, 1), jnp.int32),
        scratch_shapes=[
            pltpu.VMEM((_TM, 1), jnp.float32),
            pltpu.VMEM((_TM, 1), jnp.int32),
        ],
    )(a_bf, bt_bf, a2, b2)


def _make_sc_gather_hist(n, d, k):
    b_per_w = n // _NW
    half = b_per_w // 2
    mesh = plsc.VectorSubcoreMesh(core_axis_name="c", subcore_axis_name="s",
                                  num_cores=_NC, num_subcores=_NS)

    @functools.partial(
        pl.kernel,
        out_type=[jax.ShapeDtypeStruct((n, d), jnp.float32),
                  jax.ShapeDtypeStruct((_NW, k), jnp.float32)],
        mesh=mesh,
        scratch_types=[
            pltpu.VMEM((b_per_w,), jnp.int32),
            pltpu.VMEM((b_per_w, d), jnp.float32),
            pltpu.VMEM((k,), jnp.float32),
            pltpu.SemaphoreType.DMA,
        ],
    )
    def sc_kernel(cb_hbm, idx_hbm, zeros_hbm, quant_hbm, counts_hbm,
                  idx_v, rows_v, cnt_v, sem):
        wid = lax.axis_index("s") * _NC + lax.axis_index("c")
        base = wid * b_per_w
        pltpu.sync_copy(idx_hbm.at[pl.ds(base, b_per_w)], idx_v)
        cp0 = pltpu.async_copy(cb_hbm.at[idx_v.at[pl.ds(0, half)]],
                               rows_v.at[pl.ds(0, half)], sem)
        cp1 = pltpu.async_copy(cb_hbm.at[idx_v.at[pl.ds(half, half)]],
                               rows_v.at[pl.ds(half, half)], sem)
        cp0.wait()
        cp1.wait()
        pltpu.sync_copy(rows_v, quant_hbm.at[pl.ds(base, b_per_w)])
        pltpu.sync_copy(zeros_hbm, cnt_v)
        lane = lax.broadcasted_iota(jnp.int32, (16,), 0)
        ones = jnp.ones((16,), jnp.float32)
        for r in range(b_per_w // 16):
            v = idx_v[pl.ds(r * 16, 16)]
            for jj in range(16):
                plsc.addupdate_scatter(cnt_v, [v], ones, mask=lane == jj)
        pltpu.sync_copy(cnt_v, counts_hbm.at[wid])

    return sc_kernel


def _stats_body(cnt_ref, perp_ref, use_ref, *, n_tokens, k):
    c = jnp.sum(cnt_ref[...], axis=0, keepdims=True)
    p = c / jnp.float32(n_tokens)
    ent = -jnp.sum(p * jnp.log(p + 1e-10))
    perp_ref[...] = jnp.reshape(jnp.exp(ent), (1, 1))
    used = jnp.sum((c > 0.0).astype(jnp.float32))
    use_ref[...] = jnp.reshape(used / jnp.float32(k), (1, 1))


def _stats(counts, n_tokens):
    nw, k = counts.shape
    body = functools.partial(_stats_body, n_tokens=n_tokens, k=k)
    return pl.pallas_call(
        body,
        out_shape=[jax.ShapeDtypeStruct((1, 1), jnp.float32),
                   jax.ShapeDtypeStruct((1, 1), jnp.float32)],
    )(counts)


def kernel(x, codebook):
    b, t, d = x.shape
    k = codebook.shape[0]
    n = b * t

    xf = jnp.reshape(x, (-1, d))
    a = xf / (jnp.linalg.norm(xf, axis=1, keepdims=True) + 1e-8)
    bn = codebook / (jnp.linalg.norm(codebook, axis=1, keepdims=True) + 1e-8)
    bn = bn / (jnp.linalg.norm(bn, axis=1, keepdims=True) + 1e-8)
    a2 = jnp.sum(a ** 2, axis=1, keepdims=True)
    b2 = jnp.sum(bn ** 2, axis=1)[None, :]
    a_bf = a.astype(jnp.bfloat16)
    bt_bf = bn.astype(jnp.bfloat16).T

    idx2d = _distance_argmin(a_bf, bt_bf, a2, b2)
    idx = idx2d[:, 0]

    zeros = jnp.zeros((k,), jnp.float32)
    quant, counts = _make_sc_gather_hist(n, d, k)(codebook, idx, zeros)
    perp, use = _stats(counts, n)

    return (jnp.reshape(quant, (b, t, d)), jnp.reshape(idx, (b, t)),
            perp[0, 0], use[0, 0])


# speedup vs baseline: 1.5786x; 1.5786x over previous
import functools

import jax
import jax.numpy as jnp
from jax import lax
from jax.experimental import pallas as pl
from jax.experimental.pallas import tpu as pltpu
from jax.experimental.pallas import tpu_sc as plsc

_TM = 512
_TK = 2048
_NC = 2
_NS = 16
_NW = _NC * _NS


def _argmin_body(a_ref, bt_ref, a2_ref, b2_ref, out_ref, minv, mini):
    j = pl.program_id(1)
    nj = pl.num_programs(1)

    @pl.when(j == 0)
    def _init():
        minv[...] = jnp.full(minv.shape, jnp.inf, jnp.float32)
        mini[...] = jnp.zeros(mini.shape, jnp.int32)

    ab = jnp.dot(a_ref[...], bt_ref[...], preferred_element_type=jnp.float32)
    d = a2_ref[...] + b2_ref[...] - 2.0 * ab
    m = jnp.min(d, axis=1, keepdims=True)
    iota = lax.broadcasted_iota(jnp.int32, d.shape, 1) + j * _TK
    il = jnp.min(jnp.where(d == m, iota, jnp.int32(2**30)), axis=1,
                 keepdims=True)
    better = m < minv[...]
    mini[...] = jnp.where(better, il, mini[...])
    minv[...] = jnp.where(better, m, minv[...])

    @pl.when(j == nj - 1)
    def _flush():
        out_ref[...] = mini[...]


def _distance_argmin(a_bf, bt_bf, a2, b2):
    n, d = a_bf.shape
    k = bt_bf.shape[1]
    grid = (n // _TM, k // _TK)
    return pl.pallas_call(
        _argmin_body,
        grid=grid,
        in_specs=[
            pl.BlockSpec((_TM, d), lambda i, j: (i, 0)),
            pl.BlockSpec((d, _TK), lambda i, j: (0, j)),
            pl.BlockSpec((_TM, 1), lambda i, j: (i, 0)),
            pl.BlockSpec((1, _TK), lambda i, j: (0, j)),
        ],
        out_specs=pl.BlockSpec((_TM, 1), lambda i, j: (i, 0)),
        out_shape=jax.ShapeDtypeStruct((n, 1), jnp.int32),
        scratch_shapes=[
            pltpu.VMEM((_TM, 1), jnp.float32),
            pltpu.VMEM((_TM, 1), jnp.int32),
        ],
    )(a_bf, bt_bf, a2, b2)


def _make_sc_gather_hist(n, d, k):
    b_per_w = n // _NW
    half = b_per_w // 2
    mesh = plsc.VectorSubcoreMesh(core_axis_name="c", subcore_axis_name="s",
                                  num_cores=_NC, num_subcores=_NS)

    @functools.partial(
        pl.kernel,
        out_type=[jax.ShapeDtypeStruct((n, d), jnp.float32),
                  jax.ShapeDtypeStruct((_NW, k), jnp.float32)],
        mesh=mesh,
        scratch_types=[
            pltpu.VMEM((b_per_w,), jnp.int32),
            pltpu.VMEM((b_per_w, d), jnp.float32),
            pltpu.VMEM((k,), jnp.float32),
            pltpu.SemaphoreType.DMA,
        ],
        compiler_params=pltpu.CompilerParams(needs_layout_passes=False),
    )
    def sc_kernel(cb_hbm, idx_hbm, zeros_hbm, quant_hbm, counts_hbm,
                  idx_v, rows_v, cnt_v, sem):
        wid = lax.axis_index("s") * _NC + lax.axis_index("c")
        base = wid * b_per_w
        pltpu.sync_copy(idx_hbm.at[pl.ds(base, b_per_w)], idx_v)
        cp0 = pltpu.async_copy(cb_hbm.at[idx_v.at[pl.ds(0, half)]],
                               rows_v.at[pl.ds(0, half)], sem)
        cp1 = pltpu.async_copy(cb_hbm.at[idx_v.at[pl.ds(half, half)]],
                               rows_v.at[pl.ds(half, half)], sem)
        cp0.wait()
        cp1.wait()
        pltpu.sync_copy(rows_v, quant_hbm.at[pl.ds(base, b_per_w)])
        pltpu.sync_copy(zeros_hbm, cnt_v)
        lane = lax.broadcasted_iota(jnp.int32, (16,), 0)
        ones = jnp.ones((16,), jnp.float32)
        for r in range(b_per_w // 16):
            v = idx_v[pl.ds(r * 16, 16)]
            for jj in range(16):
                plsc.addupdate_scatter(cnt_v, [v], ones, mask=lane == jj)
        pltpu.sync_copy(cnt_v, counts_hbm.at[wid])

    return sc_kernel


def _stats_body(cnt_ref, perp_ref, use_ref, *, n_tokens, k):
    c = jnp.sum(cnt_ref[...], axis=0, keepdims=True)
    p = c / jnp.float32(n_tokens)
    ent = -jnp.sum(p * jnp.log(p + 1e-10))
    perp_ref[...] = jnp.reshape(jnp.exp(ent), (1, 1))
    used = jnp.sum((c > 0.0).astype(jnp.float32))
    use_ref[...] = jnp.reshape(used / jnp.float32(k), (1, 1))


def _stats(counts, n_tokens):
    nw, k = counts.shape
    body = functools.partial(_stats_body, n_tokens=n_tokens, k=k)
    return pl.pallas_call(
        body,
        out_shape=[jax.ShapeDtypeStruct((1, 1), jnp.float32),
                   jax.ShapeDtypeStruct((1, 1), jnp.float32)],
    )(counts)


def kernel(x, codebook):
    b, t, d = x.shape
    k = codebook.shape[0]
    n = b * t

    xf = jnp.reshape(x, (-1, d))
    a = xf / (jnp.linalg.norm(xf, axis=1, keepdims=True) + 1e-8)
    bn = codebook / (jnp.linalg.norm(codebook, axis=1, keepdims=True) + 1e-8)
    bn = bn / (jnp.linalg.norm(bn, axis=1, keepdims=True) + 1e-8)
    a2 = jnp.sum(a ** 2, axis=1, keepdims=True)
    b2 = jnp.sum(bn ** 2, axis=1)[None, :]
    a_bf = a.astype(jnp.bfloat16)
    bt_bf = bn.astype(jnp.bfloat16).T

    idx2d = _distance_argmin(a_bf, bt_bf, a2, b2)
    idx = idx2d[:, 0]

    zeros = jnp.zeros((k,), jnp.float32)
    quant, counts = _make_sc_gather_hist(n, d, k)(codebook, idx, zeros)
    perp, use = _stats(counts, n)

    return (jnp.reshape(quant, (b, t, d)), jnp.reshape(idx, (b, t)),
            perp[0, 0], use[0, 0])


# speedup vs baseline: 1.8818x; 1.1920x over previous
import functools

import jax
import jax.numpy as jnp
from jax import lax
from jax.experimental import pallas as pl
from jax.experimental.pallas import tpu as pltpu
from jax.experimental.pallas import tpu_sc as plsc

_TM = 512
_TK = 4096
_NC = 2
_NS = 16
_NW = _NC * _NS


_LANES = 128


def _argmin_body(a_ref, b_ref, a2_ref, b2_ref, iota_ref, out_ref, minv, mini):
    j = pl.program_id(1)
    nj = pl.num_programs(1)

    @pl.when(j == 0)
    def _init():
        minv[...] = jnp.full(minv.shape, jnp.inf, jnp.float32)
        mini[...] = jnp.zeros(mini.shape, jnp.float32)

    ab = lax.dot_general(a_ref[...], b_ref[...],
                         dimension_numbers=(((1,), (1,)), ((), ())),
                         preferred_element_type=jnp.float32)
    d = a2_ref[...] + b2_ref[...] - 2.0 * ab
    m_run = minv[...]
    i_run = mini[...]
    iota_row = iota_ref[...]
    for c in range(_TK // _LANES):
        dc = d[:, c * _LANES:(c + 1) * _LANES]
        ic = iota_row[:, c * _LANES:(c + 1) * _LANES]
        mask = dc < m_run
        m_run = jnp.where(mask, dc, m_run)
        i_run = jnp.where(mask, ic, i_run)
    minv[...] = m_run
    mini[...] = i_run

    @pl.when(j == nj - 1)
    def _flush():
        m = jnp.min(m_run, axis=1, keepdims=True)
        il = jnp.min(jnp.where(m_run == m, i_run, jnp.float32(2**30)),
                     axis=1, keepdims=True)
        out_ref[...] = il.astype(jnp.int32)


def _distance_argmin(a_bf, b_bf, a2, b2, iota):
    n, d = a_bf.shape
    k = b_bf.shape[0]
    grid = (n // _TM, k // _TK)
    return pl.pallas_call(
        _argmin_body,
        grid=grid,
        in_specs=[
            pl.BlockSpec((_TM, d), lambda i, j: (i, 0)),
            pl.BlockSpec((_TK, d), lambda i, j: (j, 0)),
            pl.BlockSpec((_TM, 1), lambda i, j: (i, 0)),
            pl.BlockSpec((1, _TK), lambda i, j: (0, j)),
            pl.BlockSpec((1, _TK), lambda i, j: (0, j)),
        ],
        out_specs=pl.BlockSpec((_TM, 1), lambda i, j: (i, 0)),
        out_shape=jax.ShapeDtypeStruct((n, 1), jnp.int32),
        scratch_shapes=[
            pltpu.VMEM((_TM, _LANES), jnp.float32),
            pltpu.VMEM((_TM, _LANES), jnp.float32),
        ],
    )(a_bf, b_bf, a2, b2, iota)


def _make_sc_gather_hist(n, d, k):
    b_per_w = n // _NW
    half = b_per_w // 2
    mesh = plsc.VectorSubcoreMesh(core_axis_name="c", subcore_axis_name="s",
                                  num_cores=_NC, num_subcores=_NS)

    @functools.partial(
        pl.kernel,
        out_type=[jax.ShapeDtypeStruct((n, d), jnp.float32),
                  jax.ShapeDtypeStruct((_NW, k), jnp.float32)],
        mesh=mesh,
        scratch_types=[
            pltpu.VMEM((b_per_w,), jnp.int32),
            pltpu.VMEM((b_per_w, d), jnp.float32),
            pltpu.VMEM((k,), jnp.float32),
            pltpu.SemaphoreType.DMA,
        ],
        compiler_params=pltpu.CompilerParams(needs_layout_passes=False),
    )
    def sc_kernel(cb_hbm, idx_hbm, zeros_hbm, quant_hbm, counts_hbm,
                  idx_v, rows_v, cnt_v, sem):
        wid = lax.axis_index("s") * _NC + lax.axis_index("c")
        base = wid * b_per_w
        pltpu.sync_copy(idx_hbm.at[pl.ds(base, b_per_w)], idx_v)
        cp0 = pltpu.async_copy(cb_hbm.at[idx_v.at[pl.ds(0, half)]],
                               rows_v.at[pl.ds(0, half)], sem)
        cp1 = pltpu.async_copy(cb_hbm.at[idx_v.at[pl.ds(half, half)]],
                               rows_v.at[pl.ds(half, half)], sem)
        cp0.wait()
        cp1.wait()
        pltpu.sync_copy(rows_v, quant_hbm.at[pl.ds(base, b_per_w)])
        pltpu.sync_copy(zeros_hbm, cnt_v)
        lane = lax.broadcasted_iota(jnp.int32, (16,), 0)
        ones = jnp.ones((16,), jnp.float32)
        for r in range(b_per_w // 16):
            v = idx_v[pl.ds(r * 16, 16)]
            for jj in range(16):
                plsc.addupdate_scatter(cnt_v, [v], ones, mask=lane == jj)
        pltpu.sync_copy(cnt_v, counts_hbm.at[wid])

    return sc_kernel


def _stats_body(cnt_ref, perp_ref, use_ref, *, n_tokens, k):
    c = jnp.sum(cnt_ref[...], axis=0, keepdims=True)
    p = c / jnp.float32(n_tokens)
    ent = -jnp.sum(p * jnp.log(p + 1e-10))
    perp_ref[...] = jnp.reshape(jnp.exp(ent), (1, 1))
    used = jnp.sum((c > 0.0).astype(jnp.float32))
    use_ref[...] = jnp.reshape(used / jnp.float32(k), (1, 1))


def _stats(counts, n_tokens):
    nw, k = counts.shape
    body = functools.partial(_stats_body, n_tokens=n_tokens, k=k)
    return pl.pallas_call(
        body,
        out_shape=[jax.ShapeDtypeStruct((1, 1), jnp.float32),
                   jax.ShapeDtypeStruct((1, 1), jnp.float32)],
    )(counts)


def kernel(x, codebook):
    b, t, d = x.shape
    k = codebook.shape[0]
    n = b * t

    xf = jnp.reshape(x, (-1, d))
    a = xf / (jnp.linalg.norm(xf, axis=1, keepdims=True) + 1e-8)
    bn = codebook / (jnp.linalg.norm(codebook, axis=1, keepdims=True) + 1e-8)
    bn = bn / (jnp.linalg.norm(bn, axis=1, keepdims=True) + 1e-8)
    a2 = jnp.sum(a ** 2, axis=1, keepdims=True)
    b2 = jnp.sum(bn ** 2, axis=1)[None, :]
    a_bf = a.astype(jnp.bfloat16)
    b_bf = bn.astype(jnp.bfloat16)

    iota = jnp.arange(k, dtype=jnp.float32)[None, :]
    idx2d = _distance_argmin(a_bf, b_bf, a2, b2, iota)
    idx = idx2d[:, 0]

    zeros = jnp.zeros((k,), jnp.float32)
    quant, counts = _make_sc_gather_hist(n, d, k)(codebook, idx, zeros)
    perp, use = _stats(counts, n)

    return (jnp.reshape(quant, (b, t, d)), jnp.reshape(idx, (b, t)),
            perp[0, 0], use[0, 0])
